# baseline (speedup 1.0000x reference)
import functools

import jax
import jax.numpy as jnp
from jax import lax
from jax.experimental import pallas as pl
from jax.experimental.pallas import tpu as pltpu
from jax.experimental.pallas import tpu_sc as plsc

_N_CLS = 1000
_N_CTX = 16
_D = 512
_L = 77
_NSLOT = 7
_NCYC = _L // _NSLOT
_SCATTER_CYC = 6

_NC = 2
_NS = 16
_NW = _NC * _NS
_CPW = 32


def _body(emb, ctx, pfx, out, bufs, ctx_v, pfx_v, psm, srs, sws, ssc):
    w = lax.axis_index("s") * _NC + lax.axis_index("c")
    c0 = w * _CPW
    cw = jnp.minimum(_N_CLS - c0, _CPW)

    lanes = lax.iota(jnp.int32, 16)

    def stage_params():
        pltpu.sync_copy(ctx, ctx_v)
        pltpu.sync_copy(pfx.at[pl.ds(c0, _CPW)], pfx_v)
        pv0 = pfx_v[pl.ds(0, 16)]
        pv1 = pfx_v[pl.ds(16, 16)]
        for j in range(16):
            psm[j] = pv0[j]
            psm[j + 16] = pv1[j]

    def run(width):
        def fire_read(r, slot):
            pltpu.async_copy(emb.at[pl.ds(r * _N_CLS + c0, width)],
                             bufs[slot].at[pl.ds(0, width)], srs[slot])

        def wait_read(slot):
            pltpu.make_async_copy(emb.at[pl.ds(c0, width)],
                                  bufs[slot].at[pl.ds(0, width)],
                                  srs[slot]).wait()

        def fire_write(r, slot):
            pltpu.async_copy(bufs[slot].at[pl.ds(0, width)],
                             out.at[pl.ds(r * _N_CLS + c0, width)], sws[slot])

        def wait_write(slot):
            pltpu.make_async_copy(bufs[slot].at[pl.ds(0, width)],
                                  out.at[pl.ds(c0, width)], sws[slot]).wait()

        def fire_scatters():
            def cls(t, carry):
                p = psm[t]
                idx = (p + lanes) * _N_CLS + (c0 + t)
                pltpu.async_copy(ctx_v, out.at[idx], ssc)
                return carry

            lax.fori_loop(0, width, cls, None)

        def wait_scatters():
            def cls(t, carry):
                idx = lanes * _N_CLS
                pltpu.make_async_copy(ctx_v, out.at[idx], ssc).wait()
                return carry

            lax.fori_loop(0, width, cls, None)

        for slot in range(_NSLOT):
            fire_read(slot, slot)
        stage_params()

        def step(g, carry):
            r0 = g * _NSLOT
            for slot in range(_NSLOT):
                r = r0 + slot
                wait_read(slot)
                fire_write(r, slot)

                @pl.when(g + 1 < _NCYC)
                def _():
                    wait_write(slot)
                    fire_read(r + _NSLOT, slot)

            @pl.when(g == _SCATTER_CYC)
            def _():
                fire_scatters()

            return carry

        lax.fori_loop(0, _NCYC, step, None)
        for slot in range(_NSLOT):
            wait_write(slot)
        wait_scatters()

    @pl.when(cw >= _CPW)
    def _():
        run(_CPW)

    @pl.when(cw < _CPW)
    def _():
        run(_N_CLS - (_NW - 1) * _CPW)


@functools.partial(
    pl.kernel,
    out_type=jax.ShapeDtypeStruct((_L * _N_CLS, _D), jnp.float32),
    mesh=plsc.VectorSubcoreMesh(core_axis_name="c", subcore_axis_name="s"),
    scratch_types=(
        [pltpu.VMEM((_CPW, _D), jnp.float32) for _ in range(_NSLOT)]
        + [
            pltpu.VMEM((_N_CTX, _D), jnp.float32),
            pltpu.VMEM((_CPW,), jnp.int32),
            pltpu.SMEM((_CPW,), jnp.int32),
        ]
        + [pltpu.SemaphoreType.DMA for _ in range(2 * _NSLOT + 1)]
    ),
)
def _splice_kernel(emb, ctx, pfx, out, *scratch):
    bufs = scratch[:_NSLOT]
    ctx_v, pfx_v, psm = scratch[_NSLOT:_NSLOT + 3]
    sems = scratch[_NSLOT + 3:]
    srs = sems[:_NSLOT]
    sws = sems[_NSLOT:2 * _NSLOT]
    ssc = sems[2 * _NSLOT]
    _body(emb, ctx, pfx, out, bufs, ctx_v, pfx_v, psm, srs, sws, ssc)


def kernel(origin_text_embedding, ctx, prefix_index):
    emb_t = jnp.transpose(origin_text_embedding, (1, 0, 2))
    emb2 = emb_t.reshape(_L * _N_CLS, _D)
    pfx = jnp.pad(prefix_index, (0, _NW * _CPW - _N_CLS))
    out2 = _splice_kernel(emb2, ctx, pfx)
    return jnp.transpose(out2.reshape(_L, _N_CLS, _D), (1, 0, 2))
